# Initial kernel scaffold; baseline (speedup 1.0000x reference)
import jax
import jax.numpy as jnp
from jax.experimental import pallas as pl


def kernel(x, edge_index, ae_idx, W_lin_0, b_lin_0, W_enc_0, b_enc_0, W_dec_0, b_dec_0, W_lin_1, b_lin_1, W_enc_1, b_enc_1, W_dec_1, b_dec_1, W_lin_2, b_lin_2, W_enc_2, b_enc_2, W_dec_2, b_dec_2):
    raise NotImplementedError("write your pallas kernel here")


# speedup vs baseline: 10.4894x; 10.4894x over previous
import functools

import jax
import jax.numpy as jnp
from jax import lax
from jax.experimental import pallas as pl
from jax.experimental.pallas import tpu as pltpu
from jax.experimental.pallas import tpu_sc as plsc

N = 10000
E = 160000
D_IN = 256
NC = 40
CODE = 4

BN = 400
BE = 640
EPT = E // 32
F32 = jnp.float32
I32 = jnp.int32

_MESH = dict(core_axis_name="c", subcore_axis_name="s")


@functools.lru_cache(maxsize=None)
def _make_scatter(vc, feat_split, ch):
    n_ch = (E // 16 if feat_split else EPT) // ch

    @functools.partial(
        pl.kernel,
        out_type=jax.ShapeDtypeStruct((2 * N, vc), F32),
        mesh=plsc.VectorSubcoreMesh(**_MESH),
        scratch_types=[
            pltpu.VMEM((ch, vc), F32),
            pltpu.VMEM((ch,), I32),
            pltpu.VMEM_SHARED((N, vc), F32),
        ],
    )
    def scatter_k(vals_hbm, col_hbm, zero_hbm, out_hbm, dbuf, cbuf, acc):
        c = lax.axis_index("c")
        s = lax.axis_index("s")

        @pl.when(s == 0)
        def _():
            pltpu.sync_copy(zero_hbm, acc)

        plsc.subcore_barrier()

        if feat_split:
            ebase = c * E + s * (E // 16)
            cbase = s * (E // 16)
        else:
            ebase = (c * 16 + s) * EPT
            cbase = ebase

        def body(i, carry):
            off = i * ch
            pltpu.sync_copy(col_hbm.at[pl.ds(cbase + off, ch)], cbuf)
            pltpu.sync_copy(vals_hbm.at[pl.ds(ebase + off, ch)], dbuf)
            pltpu.sync_copy(dbuf, acc.at[cbuf], add=True)
            return carry

        lax.fori_loop(0, n_ch, body, 0)
        plsc.subcore_barrier()

        @pl.when(s == 0)
        def _():
            pltpu.sync_copy(acc, out_hbm.at[pl.ds(c * N, N)])

    return scatter_k


@functools.lru_cache(maxsize=None)
def _make_gather_tab(vc, ch=40):
    n_ch = EPT // ch

    @functools.partial(
        pl.kernel,
        out_type=jax.ShapeDtypeStruct((E, vc), F32),
        mesh=plsc.VectorSubcoreMesh(**_MESH),
        scratch_types=[
            pltpu.VMEM((ch,), I32),
            pltpu.VMEM((ch, vc), F32),
        ],
    )
    def gather_k(tab_hbm, idx_hbm, out_hbm, ibuf, gbuf):
        c = lax.axis_index("c")
        s = lax.axis_index("s")
        base = (s * 2 + c) * EPT

        def body(i, carry):
            off = base + i * ch
            pltpu.sync_copy(idx_hbm.at[pl.ds(off, ch)], ibuf)
            pltpu.sync_copy(tab_hbm.at[ibuf], gbuf)
            pltpu.sync_copy(gbuf, out_hbm.at[pl.ds(off, ch)])
            return carry

        lax.fori_loop(0, n_ch, body, 0)

    return gather_k


def _disinv_body(deg_ref, out_ref):
    deg = deg_ref[0:1, :] + deg_ref[1:2, :]
    dis = jnp.where(deg > 0.0, lax.rsqrt(jnp.maximum(deg, 1e-12)), 0.0)
    invc = 1.0 / jnp.maximum(deg, 1.0)
    out_ref[...] = jnp.concatenate([dis, invc], axis=0)


def _disinv(deg2):
    return pl.pallas_call(
        _disinv_body,
        out_shape=jax.ShapeDtypeStruct((2, N), F32),
    )(deg2)


def _dense_body(first, dout, *refs):
    if first:
        (x_ref, wl_ref, bl_ref, wef_ref, be_ref, ae_ref, dis_ref,
         h_ref, encd_ref) = refs
        xb = x_ref[...]
    else:
        (hp_ref, st_ref, ic_ref, wl_ref, bl_ref, wef_ref, be_ref, ae_ref,
         dis_ref, h_ref, encd_ref) = refs
        agg = jnp.concatenate([st_ref[0], st_ref[1]], axis=1)
        xb = jnp.maximum(hp_ref[...] + agg * ic_ref[0, 0, :][:, None], 0.0)

    h = jnp.dot(xb, wl_ref[...], preferred_element_type=F32) + bl_ref[...]
    encall = jnp.dot(h, wef_ref[...], preferred_element_type=F32)
    ae = ae_ref[0, 0, :]
    j160 = lax.broadcasted_iota(I32, (BN, NC * CODE), 1)
    m = (j160 // CODE == ae[:, None]).astype(F32)
    s4 = (lax.broadcasted_iota(I32, (NC * CODE, CODE), 0) % CODE
          == lax.broadcasted_iota(I32, (NC * CODE, CODE), 1)).astype(F32)
    o40 = (ae[:, None] == lax.broadcasted_iota(I32, (BN, NC), 1)).astype(F32)
    enc4 = (jnp.dot(encall * m, s4, preferred_element_type=F32)
            + jnp.dot(o40, be_ref[...], preferred_element_type=F32))
    encd_ref[...] = jnp.zeros((BN, 128), F32)
    encd_ref[:, :CODE] = jnp.maximum(enc4, 0.0) * dis_ref[0, 0, :][:, None]
    h_ref[...] = h


def _make_dense(first, din, dout):
    grid = (N // BN,)
    nodeblk = pl.BlockSpec((1, 1, BN), lambda i: (i, 0, 0))
    in_specs = [pl.BlockSpec((BN, din), lambda i: (i, 0))]
    if not first:
        in_specs += [pl.BlockSpec((2, BN, 128), lambda i: (0, i, 0)), nodeblk]
    in_specs += [
        pl.BlockSpec((din, dout), lambda i: (0, 0)),
        pl.BlockSpec((1, dout), lambda i: (0, 0)),
        pl.BlockSpec((dout, NC * CODE), lambda i: (0, 0)),
        pl.BlockSpec((NC, CODE), lambda i: (0, 0)),
        nodeblk,
        nodeblk,
    ]
    out_specs = [
        pl.BlockSpec((BN, dout), lambda i: (i, 0)),
        pl.BlockSpec((BN, 128), lambda i: (i, 0)),
    ]
    return pl.pallas_call(
        functools.partial(_dense_body, first, dout),
        grid=grid,
        in_specs=in_specs,
        out_specs=out_specs,
        out_shape=[
            jax.ShapeDtypeStruct((N, dout), F32),
            jax.ShapeDtypeStruct((N, 128), F32),
        ],
    )


def _dec_body(dpad, split, msg_ref, cls_ref, disc_ref, wd_ref, bd_ref, out_ref):
    cls = cls_ref[0, 0, :]
    o = (cls[:, None] == lax.broadcasted_iota(I32, (BE, NC), 1)).astype(F32)
    acc = jnp.dot(o, bd_ref[...], preferred_element_type=F32)
    msg = msg_ref[:, :CODE] * disc_ref[0, 0, :][:, None]
    for k in range(CODE):
        acc = acc + jnp.dot(o * msg[:, k][:, None], wd_ref[k],
                            preferred_element_type=F32)
    d = jnp.maximum(acc, 0.0)
    if split:
        hw = dpad // 2
        out_ref[0] = d[:, :hw]
        out_ref[1] = d[:, hw:]
    else:
        out_ref[...] = d


def _make_dec(dpad, split):
    grid = (E // BE,)
    in_specs = [
        pl.BlockSpec((BE, 128), lambda i: (i, 0)),
        pl.BlockSpec((1, 1, BE), lambda i: (i, 0, 0)),
        pl.BlockSpec((1, 1, BE), lambda i: (i, 0, 0)),
        pl.BlockSpec((CODE, NC, dpad), lambda i: (0, 0, 0)),
        pl.BlockSpec((NC, dpad), lambda i: (0, 0)),
    ]
    if split:
        out_specs = pl.BlockSpec((2, BE, dpad // 2), lambda i: (0, i, 0))
        out_shape = jax.ShapeDtypeStruct((2, E, dpad // 2), F32)
    else:
        out_specs = pl.BlockSpec((BE, dpad), lambda i: (i, 0))
        out_shape = jax.ShapeDtypeStruct((E, dpad), F32)
    return pl.pallas_call(
        functools.partial(_dec_body, dpad, split),
        grid=grid,
        in_specs=in_specs,
        out_specs=out_specs,
        out_shape=out_shape,
    )


def _final_body(h_ref, s_ref, ic_ref, out_ref):
    agg = (s_ref[0, :, :NC] + s_ref[1, :, :NC]) * ic_ref[0, 0, :][:, None]
    out_ref[...] = jnp.maximum(h_ref[...] + agg, 0.0)


def _final(h2, s2, invc3):
    grid = (N // BN,)
    return pl.pallas_call(
        _final_body,
        grid=grid,
        in_specs=[
            pl.BlockSpec((BN, NC), lambda i: (i, 0)),
            pl.BlockSpec((2, BN, 128), lambda i: (0, i, 0)),
            pl.BlockSpec((1, 1, BN), lambda i: (i, 0, 0)),
        ],
        out_specs=pl.BlockSpec((BN, NC), lambda i: (i, 0)),
        out_shape=jax.ShapeDtypeStruct((N, NC), F32),
    )(h2, s2, invc3)


def kernel(x, edge_index, ae_idx,
           W_lin_0, b_lin_0, W_enc_0, b_enc_0, W_dec_0, b_dec_0,
           W_lin_1, b_lin_1, W_enc_1, b_enc_1, W_dec_1, b_dec_1,
           W_lin_2, b_lin_2, W_enc_2, b_enc_2, W_dec_2, b_dec_2):
    row = edge_index[0]
    col = edge_index[1]
    ae3 = ae_idx.reshape(N // BN, 1, BN)

    ones_e = jnp.ones((E, 128), F32)
    zero_n128 = jnp.zeros((N, 128), F32)
    degs = _make_scatter(128, False, 40)(ones_e, col, zero_n128)
    deg2 = jnp.stack([degs[:N, 0], degs[N:, 0]])
    dv = _disinv(deg2)
    dis = dv[0]
    dis3 = dis.reshape(N // BN, 1, BN)
    invc3 = dv[1].reshape(N // BN, 1, BN)

    disae = jnp.concatenate(
        [dis[:, None], ae_idx.astype(F32)[:, None],
         jnp.zeros((N, 126), F32)], axis=1)
    g2 = _make_gather_tab(128)(disae, col)
    disc3 = g2[:, 0].reshape(E // BE, 1, BE)
    cls3 = g2[:, 1].astype(I32).reshape(E // BE, 1, BE)


    layers = [
        (W_lin_0, b_lin_0, W_enc_0, b_enc_0, W_dec_0, b_dec_0, 256),
        (W_lin_1, b_lin_1, W_enc_1, b_enc_1, W_dec_1, b_dec_1, 256),
        (W_lin_2, b_lin_2, W_enc_2, b_enc_2, W_dec_2, b_dec_2, 40),
    ]

    h_prev = None
    sT_prev = None
    out40 = None
    for li, (Wl, bl, We, be, Wd, bd, dout) in enumerate(layers):
        WlT = Wl.T
        bl2 = bl.reshape(1, dout)
        WencF = We.transpose(1, 0, 2).reshape(dout, NC * CODE)
        dpad = 256 if dout == 256 else 128
        WdT = Wd.transpose(1, 0, 2)
        if dpad != dout:
            WdT = jnp.pad(WdT, ((0, 0), (0, 0), (0, dpad - dout)))
            bdp = jnp.pad(bd, ((0, 0), (0, dpad - dout)))
        else:
            bdp = bd

        if li == 0:
            h, encd = _make_dense(True, D_IN, dout)(
                x, WlT, bl2, WencF, be, ae3, dis3)
        else:
            h, encd = _make_dense(False, D_IN, dout)(
                h_prev, sT_prev, invc3, WlT, bl2, WencF, be, ae3, dis3)

        msg = _make_gather_tab(128)(encd, row)

        if dout == 256:
            decT = _make_dec(256, True)(msg, cls3, disc3, WdT, bdp)
            sflat = _make_scatter(128, True, 80)(decT.reshape(2 * E, 128), col, zero_n128)
            sT_prev = sflat.reshape(2, N, 128)
            h_prev = h
        else:
            dec = _make_dec(128, False)(msg, cls3, disc3, WdT, bdp)
            s2 = _make_scatter(128, False, 40)(dec, col, zero_n128).reshape(2, N, 128)
            out40 = _final(h, s2, invc3)

    return out40


# speedup vs baseline: 13.8766x; 1.3229x over previous
import functools

import jax
import jax.numpy as jnp
from jax import lax
from jax.experimental import pallas as pl
from jax.experimental.pallas import tpu as pltpu
from jax.experimental.pallas import tpu_sc as plsc

N = 10000
E = 160000
D_IN = 256
NC = 40
CODE = 4

BN = 400
BE = 640
EPT = E // 32
F32 = jnp.float32
I32 = jnp.int32

_MESH = dict(core_axis_name="c", subcore_axis_name="s")


@functools.lru_cache(maxsize=None)
def _make_scatter(vc, feat_split, ch):
    ept = E // 16 if feat_split else EPT
    n_ch = ept // ch
    tail = ept - n_ch * ch

    scratch = [
        pltpu.VMEM((ch, vc), F32),
        pltpu.VMEM((ch,), I32),
        pltpu.VMEM_SHARED((N, vc), F32),
    ]
    if tail:
        scratch += [pltpu.VMEM((tail, vc), F32), pltpu.VMEM((tail,), I32)]

    @functools.partial(
        pl.kernel,
        out_type=jax.ShapeDtypeStruct((2 * N, vc), F32),
        mesh=plsc.VectorSubcoreMesh(**_MESH),
        scratch_types=scratch,
    )
    def scatter_k(vals_hbm, col_hbm, zero_hbm, out_hbm, dbuf, cbuf, acc,
                  *tailbufs):
        c = lax.axis_index("c")
        s = lax.axis_index("s")

        @pl.when(s == 0)
        def _():
            pltpu.sync_copy(zero_hbm, acc)

        plsc.subcore_barrier()

        if feat_split:
            ebase = c * E + s * ept
            cbase = s * ept
        else:
            ebase = (c * 16 + s) * ept
            cbase = ebase

        def body(i, carry):
            off = i * ch
            pltpu.sync_copy(col_hbm.at[pl.ds(cbase + off, ch)], cbuf)
            pltpu.sync_copy(vals_hbm.at[pl.ds(ebase + off, ch)], dbuf)
            pltpu.sync_copy(dbuf, acc.at[cbuf], add=True)
            return carry

        lax.fori_loop(0, n_ch, body, 0)
        if tail:
            dbuf_t, cbuf_t = tailbufs
            off = n_ch * ch
            pltpu.sync_copy(col_hbm.at[pl.ds(cbase + off, tail)], cbuf_t)
            pltpu.sync_copy(vals_hbm.at[pl.ds(ebase + off, tail)], dbuf_t)
            pltpu.sync_copy(dbuf_t, acc.at[cbuf_t], add=True)
        plsc.subcore_barrier()

        @pl.when(s == 0)
        def _():
            pltpu.sync_copy(acc, out_hbm.at[pl.ds(c * N, N)])

    return scatter_k


@functools.lru_cache(maxsize=None)
def _make_gather_tab(vc, ch=128):
    n_ch = EPT // ch
    tail = EPT - n_ch * ch

    scratch = [
        pltpu.VMEM((ch,), I32),
        pltpu.VMEM((ch, vc), F32),
    ]
    if tail:
        scratch += [pltpu.VMEM((tail,), I32), pltpu.VMEM((tail, vc), F32)]

    @functools.partial(
        pl.kernel,
        out_type=jax.ShapeDtypeStruct((E, vc), F32),
        mesh=plsc.VectorSubcoreMesh(**_MESH),
        scratch_types=scratch,
    )
    def gather_k(tab_hbm, idx_hbm, out_hbm, ibuf, gbuf, *tailbufs):
        c = lax.axis_index("c")
        s = lax.axis_index("s")
        base = (s * 2 + c) * EPT

        def body(i, carry):
            off = base + i * ch
            pltpu.sync_copy(idx_hbm.at[pl.ds(off, ch)], ibuf)
            pltpu.sync_copy(tab_hbm.at[ibuf], gbuf)
            pltpu.sync_copy(gbuf, out_hbm.at[pl.ds(off, ch)])
            return carry

        lax.fori_loop(0, n_ch, body, 0)
        if tail:
            ibuf_t, gbuf_t = tailbufs
            off = base + n_ch * ch
            pltpu.sync_copy(idx_hbm.at[pl.ds(off, tail)], ibuf_t)
            pltpu.sync_copy(tab_hbm.at[ibuf_t], gbuf_t)
            pltpu.sync_copy(gbuf_t, out_hbm.at[pl.ds(off, tail)])

    return gather_k


def _disinv_body(deg_ref, out_ref):
    deg = deg_ref[0:1, :] + deg_ref[1:2, :]
    dis = jnp.where(deg > 0.0, lax.rsqrt(jnp.maximum(deg, 1e-12)), 0.0)
    invc = 1.0 / jnp.maximum(deg, 1.0)
    out_ref[...] = jnp.concatenate([dis, invc], axis=0)


def _disinv(deg2):
    return pl.pallas_call(
        _disinv_body,
        out_shape=jax.ShapeDtypeStruct((2, N), F32),
    )(deg2)


def _dense_body(first, dout, *refs):
    if first:
        (x_ref, wl_ref, bl_ref, wef_ref, be_ref, ae_ref, dis_ref,
         h_ref, encd_ref) = refs
        xb = x_ref[...]
    else:
        (hp_ref, st_ref, ic_ref, wl_ref, bl_ref, wef_ref, be_ref, ae_ref,
         dis_ref, h_ref, encd_ref) = refs
        agg = jnp.concatenate([st_ref[0], st_ref[1]], axis=1)
        xb = jnp.maximum(hp_ref[...] + agg * ic_ref[0, 0, :][:, None], 0.0)

    h = jnp.dot(xb, wl_ref[...], preferred_element_type=F32) + bl_ref[...]
    encall = jnp.dot(h, wef_ref[...], preferred_element_type=F32)
    ae = ae_ref[0, 0, :]
    j160 = lax.broadcasted_iota(I32, (BN, NC * CODE), 1)
    m = (j160 // CODE == ae[:, None]).astype(F32)
    s4 = (lax.broadcasted_iota(I32, (NC * CODE, CODE), 0) % CODE
          == lax.broadcasted_iota(I32, (NC * CODE, CODE), 1)).astype(F32)
    o40 = (ae[:, None] == lax.broadcasted_iota(I32, (BN, NC), 1)).astype(F32)
    enc4 = (jnp.dot(encall * m, s4, preferred_element_type=F32)
            + jnp.dot(o40, be_ref[...], preferred_element_type=F32))
    encd_ref[...] = jnp.zeros((BN, 128), F32)
    encd_ref[:, :CODE] = jnp.maximum(enc4, 0.0) * dis_ref[0, 0, :][:, None]
    h_ref[...] = h


def _make_dense(first, din, dout):
    grid = (N // BN,)
    nodeblk = pl.BlockSpec((1, 1, BN), lambda i: (i, 0, 0))
    in_specs = [pl.BlockSpec((BN, din), lambda i: (i, 0))]
    if not first:
        in_specs += [pl.BlockSpec((2, BN, 128), lambda i: (0, i, 0)), nodeblk]
    in_specs += [
        pl.BlockSpec((din, dout), lambda i: (0, 0)),
        pl.BlockSpec((1, dout), lambda i: (0, 0)),
        pl.BlockSpec((dout, NC * CODE), lambda i: (0, 0)),
        pl.BlockSpec((NC, CODE), lambda i: (0, 0)),
        nodeblk,
        nodeblk,
    ]
    out_specs = [
        pl.BlockSpec((BN, dout), lambda i: (i, 0)),
        pl.BlockSpec((BN, 128), lambda i: (i, 0)),
    ]
    return pl.pallas_call(
        functools.partial(_dense_body, first, dout),
        grid=grid,
        in_specs=in_specs,
        out_specs=out_specs,
        out_shape=[
            jax.ShapeDtypeStruct((N, dout), F32),
            jax.ShapeDtypeStruct((N, 128), F32),
        ],
    )


def _dec_body(dpad, split, msg_ref, cls_ref, disc_ref, wd_ref, bd_ref, out_ref):
    cls = cls_ref[0, 0, :]
    o = (cls[:, None] == lax.broadcasted_iota(I32, (BE, NC), 1)).astype(F32)
    acc = jnp.dot(o, bd_ref[...], preferred_element_type=F32)
    msg = msg_ref[:, :CODE] * disc_ref[0, 0, :][:, None]
    for k in range(CODE):
        acc = acc + jnp.dot(o * msg[:, k][:, None], wd_ref[k],
                            preferred_element_type=F32)
    d = jnp.maximum(acc, 0.0)
    if split:
        hw = dpad // 2
        out_ref[0] = d[:, :hw]
        out_ref[1] = d[:, hw:]
    else:
        out_ref[...] = d


def _make_dec(dpad, split):
    grid = (E // BE,)
    in_specs = [
        pl.BlockSpec((BE, 128), lambda i: (i, 0)),
        pl.BlockSpec((1, 1, BE), lambda i: (i, 0, 0)),
        pl.BlockSpec((1, 1, BE), lambda i: (i, 0, 0)),
        pl.BlockSpec((CODE, NC, dpad), lambda i: (0, 0, 0)),
        pl.BlockSpec((NC, dpad), lambda i: (0, 0)),
    ]
    if split:
        out_specs = pl.BlockSpec((2, BE, dpad // 2), lambda i: (0, i, 0))
        out_shape = jax.ShapeDtypeStruct((2, E, dpad // 2), F32)
    else:
        out_specs = pl.BlockSpec((BE, dpad), lambda i: (i, 0))
        out_shape = jax.ShapeDtypeStruct((E, dpad), F32)
    return pl.pallas_call(
        functools.partial(_dec_body, dpad, split),
        grid=grid,
        in_specs=in_specs,
        out_specs=out_specs,
        out_shape=out_shape,
    )


def _final_body(h_ref, s_ref, ic_ref, out_ref):
    agg = (s_ref[0, :, :NC] + s_ref[1, :, :NC]) * ic_ref[0, 0, :][:, None]
    out_ref[...] = jnp.maximum(h_ref[...] + agg, 0.0)


def _final(h2, s2, invc3):
    grid = (N // BN,)
    return pl.pallas_call(
        _final_body,
        grid=grid,
        in_specs=[
            pl.BlockSpec((BN, NC), lambda i: (i, 0)),
            pl.BlockSpec((2, BN, 128), lambda i: (0, i, 0)),
            pl.BlockSpec((1, 1, BN), lambda i: (i, 0, 0)),
        ],
        out_specs=pl.BlockSpec((BN, NC), lambda i: (i, 0)),
        out_shape=jax.ShapeDtypeStruct((N, NC), F32),
    )(h2, s2, invc3)


def kernel(x, edge_index, ae_idx,
           W_lin_0, b_lin_0, W_enc_0, b_enc_0, W_dec_0, b_dec_0,
           W_lin_1, b_lin_1, W_enc_1, b_enc_1, W_dec_1, b_dec_1,
           W_lin_2, b_lin_2, W_enc_2, b_enc_2, W_dec_2, b_dec_2):
    row = edge_index[0]
    col = edge_index[1]
    ae3 = ae_idx.reshape(N // BN, 1, BN)

    ones_e = jnp.ones((E, 128), F32)
    zero_n128 = jnp.zeros((N, 128), F32)
    degs = _make_scatter(128, False, 128)(ones_e, col, zero_n128)
    deg2 = jnp.stack([degs[:N, 0], degs[N:, 0]])
    dv = _disinv(deg2)
    dis = dv[0]
    dis3 = dis.reshape(N // BN, 1, BN)
    invc3 = dv[1].reshape(N // BN, 1, BN)

    disae = jnp.concatenate(
        [dis[:, None], ae_idx.astype(F32)[:, None],
         jnp.zeros((N, 126), F32)], axis=1)
    g2 = _make_gather_tab(128)(disae, col)
    disc3 = g2[:, 0].reshape(E // BE, 1, BE)
    cls3 = g2[:, 1].astype(I32).reshape(E // BE, 1, BE)


    layers = [
        (W_lin_0, b_lin_0, W_enc_0, b_enc_0, W_dec_0, b_dec_0, 256),
        (W_lin_1, b_lin_1, W_enc_1, b_enc_1, W_dec_1, b_dec_1, 256),
        (W_lin_2, b_lin_2, W_enc_2, b_enc_2, W_dec_2, b_dec_2, 40),
    ]

    h_prev = None
    sT_prev = None
    out40 = None
    for li, (Wl, bl, We, be, Wd, bd, dout) in enumerate(layers):
        WlT = Wl.T
        bl2 = bl.reshape(1, dout)
        WencF = We.transpose(1, 0, 2).reshape(dout, NC * CODE)
        dpad = 256 if dout == 256 else 128
        WdT = Wd.transpose(1, 0, 2)
        if dpad != dout:
            WdT = jnp.pad(WdT, ((0, 0), (0, 0), (0, dpad - dout)))
            bdp = jnp.pad(bd, ((0, 0), (0, dpad - dout)))
        else:
            bdp = bd

        if li == 0:
            h, encd = _make_dense(True, D_IN, dout)(
                x, WlT, bl2, WencF, be, ae3, dis3)
        else:
            h, encd = _make_dense(False, D_IN, dout)(
                h_prev, sT_prev, invc3, WlT, bl2, WencF, be, ae3, dis3)

        msg = _make_gather_tab(128)(encd, row)

        if dout == 256:
            decT = _make_dec(256, True)(msg, cls3, disc3, WdT, bdp)
            sflat = _make_scatter(128, True, 128)(decT.reshape(2 * E, 128), col, zero_n128)
            sT_prev = sflat.reshape(2, N, 128)
            h_prev = h
        else:
            dec = _make_dec(128, False)(msg, cls3, disc3, WdT, bdp)
            s2 = _make_scatter(128, False, 128)(dec, col, zero_n128).reshape(2, N, 128)
            out40 = _final(h, s2, invc3)

    return out40


# speedup vs baseline: 14.4038x; 1.0380x over previous
import functools

import jax
import jax.numpy as jnp
from jax import lax
from jax.experimental import pallas as pl
from jax.experimental.pallas import tpu as pltpu
from jax.experimental.pallas import tpu_sc as plsc

N = 10000
E = 160000
D_IN = 256
NC = 40
CODE = 4

BN = 400
BE = 640
EPT = E // 32
F32 = jnp.float32
I32 = jnp.int32

_MESH = dict(core_axis_name="c", subcore_axis_name="s")


@functools.lru_cache(maxsize=None)
def _make_scatter(vc, feat_split, ch):
    ept = E // 16 if feat_split else EPT
    n_ch = ept // ch
    tail = ept - n_ch * ch

    scratch = [
        pltpu.VMEM((ch, vc), F32),
        pltpu.VMEM((ch,), I32),
        pltpu.VMEM_SHARED((N, vc), F32),
    ]
    if tail:
        scratch += [pltpu.VMEM((tail, vc), F32), pltpu.VMEM((tail,), I32)]

    @functools.partial(
        pl.kernel,
        out_type=jax.ShapeDtypeStruct((2 * N, vc), F32),
        mesh=plsc.VectorSubcoreMesh(**_MESH),
        scratch_types=scratch,
    )
    def scatter_k(vals_hbm, col_hbm, zero_hbm, out_hbm, dbuf, cbuf, acc,
                  *tailbufs):
        c = lax.axis_index("c")
        s = lax.axis_index("s")

        @pl.when(s == 0)
        def _():
            pltpu.sync_copy(zero_hbm, acc)

        plsc.subcore_barrier()

        if feat_split:
            ebase = c * E + s * ept
            cbase = s * ept
        else:
            ebase = (c * 16 + s) * ept
            cbase = ebase

        def body(i, carry):
            off = i * ch
            pltpu.sync_copy(col_hbm.at[pl.ds(cbase + off, ch)], cbuf)
            pltpu.sync_copy(vals_hbm.at[pl.ds(ebase + off, ch)], dbuf)
            pltpu.sync_copy(dbuf, acc.at[cbuf], add=True)
            return carry

        lax.fori_loop(0, n_ch, body, 0)
        if tail:
            dbuf_t, cbuf_t = tailbufs
            off = n_ch * ch
            pltpu.sync_copy(col_hbm.at[pl.ds(cbase + off, tail)], cbuf_t)
            pltpu.sync_copy(vals_hbm.at[pl.ds(ebase + off, tail)], dbuf_t)
            pltpu.sync_copy(dbuf_t, acc.at[cbuf_t], add=True)
        plsc.subcore_barrier()

        @pl.when(s == 0)
        def _():
            pltpu.sync_copy(acc, out_hbm.at[pl.ds(c * N, N)])

    return scatter_k


@functools.lru_cache(maxsize=None)
def _make_gather_tab(vc, ch=128):
    n_ch = EPT // ch
    tail = EPT - n_ch * ch

    scratch = [
        pltpu.VMEM((EPT,), I32),
        pltpu.VMEM((ch, vc), F32),
    ]
    if tail:
        scratch += [pltpu.VMEM((tail, vc), F32)]

    @functools.partial(
        pl.kernel,
        out_type=jax.ShapeDtypeStruct((E, vc), F32),
        mesh=plsc.VectorSubcoreMesh(**_MESH),
        scratch_types=scratch,
    )
    def gather_k(tab_hbm, idx_hbm, out_hbm, ibuf, gbuf, *tailbufs):
        c = lax.axis_index("c")
        s = lax.axis_index("s")
        base = (s * 2 + c) * EPT
        pltpu.sync_copy(idx_hbm.at[pl.ds(base, EPT)], ibuf)

        def body(i, carry):
            off = i * ch
            pltpu.sync_copy(tab_hbm.at[ibuf.at[pl.ds(off, ch)]], gbuf)
            pltpu.sync_copy(gbuf, out_hbm.at[pl.ds(base + off, ch)])
            return carry

        lax.fori_loop(0, n_ch, body, 0)
        if tail:
            (gbuf_t,) = tailbufs
            off = n_ch * ch
            pltpu.sync_copy(tab_hbm.at[ibuf.at[pl.ds(off, tail)]], gbuf_t)
            pltpu.sync_copy(gbuf_t, out_hbm.at[pl.ds(base + off, tail)])

    return gather_k


def _disinv_body(deg_ref, out_ref):
    deg = deg_ref[0:1, :] + deg_ref[1:2, :]
    dis = jnp.where(deg > 0.0, lax.rsqrt(jnp.maximum(deg, 1e-12)), 0.0)
    invc = 1.0 / jnp.maximum(deg, 1.0)
    out_ref[...] = jnp.concatenate([dis, invc], axis=0)


def _disinv(deg2):
    return pl.pallas_call(
        _disinv_body,
        out_shape=jax.ShapeDtypeStruct((2, N), F32),
    )(deg2)


def _dense_body(first, dout, *refs):
    if first:
        (x_ref, wl_ref, bl_ref, wef_ref, be_ref, ae_ref, dis_ref,
         h_ref, encd_ref) = refs
        xb = x_ref[...]
    else:
        (hp_ref, st_ref, ic_ref, wl_ref, bl_ref, wef_ref, be_ref, ae_ref,
         dis_ref, h_ref, encd_ref) = refs
        agg = jnp.concatenate([st_ref[0], st_ref[1]], axis=1)
        xb = jnp.maximum(hp_ref[...] + agg * ic_ref[0, 0, :][:, None], 0.0)

    h = jnp.dot(xb, wl_ref[...], preferred_element_type=F32) + bl_ref[...]
    encall = jnp.dot(h, wef_ref[...], preferred_element_type=F32)
    ae = ae_ref[0, 0, :]
    j160 = lax.broadcasted_iota(I32, (BN, NC * CODE), 1)
    m = (j160 // CODE == ae[:, None]).astype(F32)
    s4 = (lax.broadcasted_iota(I32, (NC * CODE, CODE), 0) % CODE
          == lax.broadcasted_iota(I32, (NC * CODE, CODE), 1)).astype(F32)
    o40 = (ae[:, None] == lax.broadcasted_iota(I32, (BN, NC), 1)).astype(F32)
    enc4 = (jnp.dot(encall * m, s4, preferred_element_type=F32)
            + jnp.dot(o40, be_ref[...], preferred_element_type=F32))
    encd_ref[...] = jnp.zeros((BN, 128), F32)
    encd_ref[:, :CODE] = jnp.maximum(enc4, 0.0) * dis_ref[0, 0, :][:, None]
    h_ref[...] = h


def _make_dense(first, din, dout):
    grid = (N // BN,)
    nodeblk = pl.BlockSpec((1, 1, BN), lambda i: (i, 0, 0))
    in_specs = [pl.BlockSpec((BN, din), lambda i: (i, 0))]
    if not first:
        in_specs += [pl.BlockSpec((2, BN, 128), lambda i: (0, i, 0)), nodeblk]
    in_specs += [
        pl.BlockSpec((din, dout), lambda i: (0, 0)),
        pl.BlockSpec((1, dout), lambda i: (0, 0)),
        pl.BlockSpec((dout, NC * CODE), lambda i: (0, 0)),
        pl.BlockSpec((NC, CODE), lambda i: (0, 0)),
        nodeblk,
        nodeblk,
    ]
    out_specs = [
        pl.BlockSpec((BN, dout), lambda i: (i, 0)),
        pl.BlockSpec((BN, 128), lambda i: (i, 0)),
    ]
    return pl.pallas_call(
        functools.partial(_dense_body, first, dout),
        grid=grid,
        in_specs=in_specs,
        out_specs=out_specs,
        out_shape=[
            jax.ShapeDtypeStruct((N, dout), F32),
            jax.ShapeDtypeStruct((N, 128), F32),
        ],
    )


def _dec_body(dpad, split, msg_ref, cls_ref, disc_ref, wd_ref, bd_ref, out_ref):
    cls = cls_ref[0, 0, :]
    o = (cls[:, None] == lax.broadcasted_iota(I32, (BE, NC), 1)).astype(F32)
    acc = jnp.dot(o, bd_ref[...], preferred_element_type=F32)
    msg = msg_ref[:, :CODE] * disc_ref[0, 0, :][:, None]
    for k in range(CODE):
        acc = acc + jnp.dot(o * msg[:, k][:, None], wd_ref[k],
                            preferred_element_type=F32)
    d = jnp.maximum(acc, 0.0)
    if split:
        hw = dpad // 2
        out_ref[0] = d[:, :hw]
        out_ref[1] = d[:, hw:]
    else:
        out_ref[...] = d


def _make_dec(dpad, split):
    grid = (E // BE,)
    in_specs = [
        pl.BlockSpec((BE, 128), lambda i: (i, 0)),
        pl.BlockSpec((1, 1, BE), lambda i: (i, 0, 0)),
        pl.BlockSpec((1, 1, BE), lambda i: (i, 0, 0)),
        pl.BlockSpec((CODE, NC, dpad), lambda i: (0, 0, 0)),
        pl.BlockSpec((NC, dpad), lambda i: (0, 0)),
    ]
    if split:
        out_specs = pl.BlockSpec((2, BE, dpad // 2), lambda i: (0, i, 0))
        out_shape = jax.ShapeDtypeStruct((2, E, dpad // 2), F32)
    else:
        out_specs = pl.BlockSpec((BE, dpad), lambda i: (i, 0))
        out_shape = jax.ShapeDtypeStruct((E, dpad), F32)
    return pl.pallas_call(
        functools.partial(_dec_body, dpad, split),
        grid=grid,
        in_specs=in_specs,
        out_specs=out_specs,
        out_shape=out_shape,
    )


def _final_body(h_ref, s_ref, ic_ref, out_ref):
    agg = (s_ref[0, :, :NC] + s_ref[1, :, :NC]) * ic_ref[0, 0, :][:, None]
    out_ref[...] = jnp.maximum(h_ref[...] + agg, 0.0)


def _final(h2, s2, invc3):
    grid = (N // BN,)
    return pl.pallas_call(
        _final_body,
        grid=grid,
        in_specs=[
            pl.BlockSpec((BN, NC), lambda i: (i, 0)),
            pl.BlockSpec((2, BN, 128), lambda i: (0, i, 0)),
            pl.BlockSpec((1, 1, BN), lambda i: (i, 0, 0)),
        ],
        out_specs=pl.BlockSpec((BN, NC), lambda i: (i, 0)),
        out_shape=jax.ShapeDtypeStruct((N, NC), F32),
    )(h2, s2, invc3)


def kernel(x, edge_index, ae_idx,
           W_lin_0, b_lin_0, W_enc_0, b_enc_0, W_dec_0, b_dec_0,
           W_lin_1, b_lin_1, W_enc_1, b_enc_1, W_dec_1, b_dec_1,
           W_lin_2, b_lin_2, W_enc_2, b_enc_2, W_dec_2, b_dec_2):
    row = edge_index[0]
    col = edge_index[1]
    ae3 = ae_idx.reshape(N // BN, 1, BN)

    ones_e = jnp.ones((E, 128), F32)
    zero_n128 = jnp.zeros((N, 128), F32)
    degs = _make_scatter(128, False, 128)(ones_e, col, zero_n128)
    deg2 = jnp.stack([degs[:N, 0], degs[N:, 0]])
    dv = _disinv(deg2)
    dis = dv[0]
    dis3 = dis.reshape(N // BN, 1, BN)
    invc3 = dv[1].reshape(N // BN, 1, BN)

    disae = jnp.concatenate(
        [dis[:, None], ae_idx.astype(F32)[:, None],
         jnp.zeros((N, 126), F32)], axis=1)
    g2 = _make_gather_tab(128)(disae, col)
    disc3 = g2[:, 0].reshape(E // BE, 1, BE)
    cls3 = g2[:, 1].astype(I32).reshape(E // BE, 1, BE)


    layers = [
        (W_lin_0, b_lin_0, W_enc_0, b_enc_0, W_dec_0, b_dec_0, 256),
        (W_lin_1, b_lin_1, W_enc_1, b_enc_1, W_dec_1, b_dec_1, 256),
        (W_lin_2, b_lin_2, W_enc_2, b_enc_2, W_dec_2, b_dec_2, 40),
    ]

    h_prev = None
    sT_prev = None
    out40 = None
    for li, (Wl, bl, We, be, Wd, bd, dout) in enumerate(layers):
        WlT = Wl.T
        bl2 = bl.reshape(1, dout)
        WencF = We.transpose(1, 0, 2).reshape(dout, NC * CODE)
        dpad = 256 if dout == 256 else 128
        WdT = Wd.transpose(1, 0, 2)
        if dpad != dout:
            WdT = jnp.pad(WdT, ((0, 0), (0, 0), (0, dpad - dout)))
            bdp = jnp.pad(bd, ((0, 0), (0, dpad - dout)))
        else:
            bdp = bd

        if li == 0:
            h, encd = _make_dense(True, D_IN, dout)(
                x, WlT, bl2, WencF, be, ae3, dis3)
        else:
            h, encd = _make_dense(False, D_IN, dout)(
                h_prev, sT_prev, invc3, WlT, bl2, WencF, be, ae3, dis3)

        msg = _make_gather_tab(128)(encd, row)

        if dout == 256:
            decT = _make_dec(256, True)(msg, cls3, disc3, WdT, bdp)
            sflat = _make_scatter(128, True, 128)(decT.reshape(2 * E, 128), col, zero_n128)
            sT_prev = sflat.reshape(2, N, 128)
            h_prev = h
        else:
            dec = _make_dec(128, False)(msg, cls3, disc3, WdT, bdp)
            s2 = _make_scatter(128, False, 128)(dec, col, zero_n128).reshape(2, N, 128)
            out40 = _final(h, s2, invc3)

    return out40


# speedup vs baseline: 15.1669x; 1.0530x over previous
import functools

import jax
import jax.numpy as jnp
from jax import lax
from jax.experimental import pallas as pl
from jax.experimental.pallas import tpu as pltpu
from jax.experimental.pallas import tpu_sc as plsc

N = 10000
E = 160000
D_IN = 256
NC = 40
CODE = 4

BN = 400
BE = 640
EPT = E // 32
F32 = jnp.float32
I32 = jnp.int32

_MESH = dict(core_axis_name="c", subcore_axis_name="s")


@functools.lru_cache(maxsize=None)
def _make_scatter(vc, feat_split, ch):
    del ch
    NR = E // 128
    maxr = 80 if feat_split else 40

    @functools.partial(
        pl.kernel,
        out_type=jax.ShapeDtypeStruct((2 * N, vc), F32),
        mesh=plsc.VectorSubcoreMesh(**_MESH),
        scratch_types=[
            pltpu.VMEM((128, vc), F32),
            pltpu.VMEM((maxr, 128), I32),
            pltpu.VMEM_SHARED((N, vc), F32),
        ],
    )
    def scatter_k(vals_hbm, col2d_hbm, zero_hbm, out_hbm, dbuf, cbuf2, acc):
        c = lax.axis_index("c")
        s = lax.axis_index("s")

        @pl.when(s == 0)
        def _():
            pltpu.sync_copy(zero_hbm, acc)

        plsc.subcore_barrier()

        w = s if feat_split else c * 16 + s
        base_row = w * maxr
        nr = jnp.minimum(maxr, NR - base_row)
        pltpu.sync_copy(col2d_hbm.at[pl.ds(base_row, maxr)], cbuf2)
        ebase = c * E if feat_split else 0

        def body(j, carry):
            off_e = ebase + (base_row + j) * 128
            pltpu.sync_copy(vals_hbm.at[pl.ds(off_e, 128)], dbuf)
            pltpu.sync_copy(dbuf, acc.at[cbuf2.at[j]], add=True)
            return carry

        lax.fori_loop(0, nr, body, 0)
        plsc.subcore_barrier()

        @pl.when(s == 0)
        def _():
            pltpu.sync_copy(acc, out_hbm.at[pl.ds(c * N, N)])

    return scatter_k


@functools.lru_cache(maxsize=None)
def _make_gather_tab(vc, ch=128):
    n_ch = EPT // ch
    tail = EPT - n_ch * ch

    scratch = [
        pltpu.VMEM((EPT,), I32),
        pltpu.VMEM((ch, vc), F32),
    ]
    if tail:
        scratch += [pltpu.VMEM((tail, vc), F32)]

    @functools.partial(
        pl.kernel,
        out_type=jax.ShapeDtypeStruct((E, vc), F32),
        mesh=plsc.VectorSubcoreMesh(**_MESH),
        scratch_types=scratch,
    )
    def gather_k(tab_hbm, idx_hbm, out_hbm, ibuf, gbuf, *tailbufs):
        c = lax.axis_index("c")
        s = lax.axis_index("s")
        base = (s * 2 + c) * EPT
        pltpu.sync_copy(idx_hbm.at[pl.ds(base, EPT)], ibuf)

        def body(i, carry):
            off = i * ch
            pltpu.sync_copy(tab_hbm.at[ibuf.at[pl.ds(off, ch)]], gbuf)
            pltpu.sync_copy(gbuf, out_hbm.at[pl.ds(base + off, ch)])
            return carry

        lax.fori_loop(0, n_ch, body, 0)
        if tail:
            (gbuf_t,) = tailbufs
            off = n_ch * ch
            pltpu.sync_copy(tab_hbm.at[ibuf.at[pl.ds(off, tail)]], gbuf_t)
            pltpu.sync_copy(gbuf_t, out_hbm.at[pl.ds(base + off, tail)])

    return gather_k


def _disinv_body(deg_ref, out_ref):
    deg = deg_ref[0:1, :] + deg_ref[1:2, :]
    dis = jnp.where(deg > 0.0, lax.rsqrt(jnp.maximum(deg, 1e-12)), 0.0)
    invc = 1.0 / jnp.maximum(deg, 1.0)
    out_ref[...] = jnp.concatenate([dis, invc], axis=0)


def _disinv(deg2):
    return pl.pallas_call(
        _disinv_body,
        out_shape=jax.ShapeDtypeStruct((2, N), F32),
    )(deg2)


def _dense_body(first, dout, *refs):
    if first:
        (x_ref, wl_ref, bl_ref, wef_ref, be_ref, ae_ref, dis_ref,
         h_ref, encd_ref) = refs
        xb = x_ref[...]
    else:
        (hp_ref, st_ref, ic_ref, wl_ref, bl_ref, wef_ref, be_ref, ae_ref,
         dis_ref, h_ref, encd_ref) = refs
        agg = jnp.concatenate([st_ref[0], st_ref[1]], axis=1)
        xb = jnp.maximum(hp_ref[...] + agg * ic_ref[0, 0, :][:, None], 0.0)

    h = jnp.dot(xb, wl_ref[...], preferred_element_type=F32) + bl_ref[...]
    encall = jnp.dot(h, wef_ref[...], preferred_element_type=F32)
    ae = ae_ref[0, 0, :]
    j160 = lax.broadcasted_iota(I32, (BN, NC * CODE), 1)
    m = (j160 // CODE == ae[:, None]).astype(F32)
    s4 = (lax.broadcasted_iota(I32, (NC * CODE, CODE), 0) % CODE
          == lax.broadcasted_iota(I32, (NC * CODE, CODE), 1)).astype(F32)
    o40 = (ae[:, None] == lax.broadcasted_iota(I32, (BN, NC), 1)).astype(F32)
    enc4 = (jnp.dot(encall * m, s4, preferred_element_type=F32)
            + jnp.dot(o40, be_ref[...], preferred_element_type=F32))
    encd_ref[...] = jnp.zeros((BN, 128), F32)
    encd_ref[:, :CODE] = jnp.maximum(enc4, 0.0) * dis_ref[0, 0, :][:, None]
    h_ref[...] = h


def _make_dense(first, din, dout):
    grid = (N // BN,)
    nodeblk = pl.BlockSpec((1, 1, BN), lambda i: (i, 0, 0))
    in_specs = [pl.BlockSpec((BN, din), lambda i: (i, 0))]
    if not first:
        in_specs += [pl.BlockSpec((2, BN, 128), lambda i: (0, i, 0)), nodeblk]
    in_specs += [
        pl.BlockSpec((din, dout), lambda i: (0, 0)),
        pl.BlockSpec((1, dout), lambda i: (0, 0)),
        pl.BlockSpec((dout, NC * CODE), lambda i: (0, 0)),
        pl.BlockSpec((NC, CODE), lambda i: (0, 0)),
        nodeblk,
        nodeblk,
    ]
    out_specs = [
        pl.BlockSpec((BN, dout), lambda i: (i, 0)),
        pl.BlockSpec((BN, 128), lambda i: (i, 0)),
    ]
    return pl.pallas_call(
        functools.partial(_dense_body, first, dout),
        grid=grid,
        in_specs=in_specs,
        out_specs=out_specs,
        out_shape=[
            jax.ShapeDtypeStruct((N, dout), F32),
            jax.ShapeDtypeStruct((N, 128), F32),
        ],
    )


def _dec_body(dpad, split, msg_ref, cls_ref, disc_ref, wd_ref, bd_ref, out_ref):
    cls = cls_ref[0, 0, :]
    o = (cls[:, None] == lax.broadcasted_iota(I32, (BE, NC), 1)).astype(F32)
    acc = jnp.dot(o, bd_ref[...], preferred_element_type=F32)
    msg = msg_ref[:, :CODE] * disc_ref[0, 0, :][:, None]
    for k in range(CODE):
        acc = acc + jnp.dot(o * msg[:, k][:, None], wd_ref[k],
                            preferred_element_type=F32)
    d = jnp.maximum(acc, 0.0)
    if split:
        hw = dpad // 2
        out_ref[0] = d[:, :hw]
        out_ref[1] = d[:, hw:]
    else:
        out_ref[...] = d


def _make_dec(dpad, split):
    grid = (E // BE,)
    in_specs = [
        pl.BlockSpec((BE, 128), lambda i: (i, 0)),
        pl.BlockSpec((1, 1, BE), lambda i: (i, 0, 0)),
        pl.BlockSpec((1, 1, BE), lambda i: (i, 0, 0)),
        pl.BlockSpec((CODE, NC, dpad), lambda i: (0, 0, 0)),
        pl.BlockSpec((NC, dpad), lambda i: (0, 0)),
    ]
    if split:
        out_specs = pl.BlockSpec((2, BE, dpad // 2), lambda i: (0, i, 0))
        out_shape = jax.ShapeDtypeStruct((2, E, dpad // 2), F32)
    else:
        out_specs = pl.BlockSpec((BE, dpad), lambda i: (i, 0))
        out_shape = jax.ShapeDtypeStruct((E, dpad), F32)
    return pl.pallas_call(
        functools.partial(_dec_body, dpad, split),
        grid=grid,
        in_specs=in_specs,
        out_specs=out_specs,
        out_shape=out_shape,
    )


def _final_body(h_ref, s_ref, ic_ref, out_ref):
    agg = (s_ref[0, :, :NC] + s_ref[1, :, :NC]) * ic_ref[0, 0, :][:, None]
    out_ref[...] = jnp.maximum(h_ref[...] + agg, 0.0)


def _final(h2, s2, invc3):
    grid = (N // BN,)
    return pl.pallas_call(
        _final_body,
        grid=grid,
        in_specs=[
            pl.BlockSpec((BN, NC), lambda i: (i, 0)),
            pl.BlockSpec((2, BN, 128), lambda i: (0, i, 0)),
            pl.BlockSpec((1, 1, BN), lambda i: (i, 0, 0)),
        ],
        out_specs=pl.BlockSpec((BN, NC), lambda i: (i, 0)),
        out_shape=jax.ShapeDtypeStruct((N, NC), F32),
    )(h2, s2, invc3)


def kernel(x, edge_index, ae_idx,
           W_lin_0, b_lin_0, W_enc_0, b_enc_0, W_dec_0, b_dec_0,
           W_lin_1, b_lin_1, W_enc_1, b_enc_1, W_dec_1, b_dec_1,
           W_lin_2, b_lin_2, W_enc_2, b_enc_2, W_dec_2, b_dec_2):
    row = edge_index[0]
    col = edge_index[1]
    ae3 = ae_idx.reshape(N // BN, 1, BN)

    col2d = jnp.pad(col.reshape(E // 128, 128), ((0, 30), (0, 0)))
    ones_e = jnp.ones((E, 128), F32)
    zero_n128 = jnp.zeros((N, 128), F32)
    degs = _make_scatter(128, False, 128)(ones_e, col2d, zero_n128)
    deg2 = jnp.stack([degs[:N, 0], degs[N:, 0]])
    dv = _disinv(deg2)
    dis = dv[0]
    dis3 = dis.reshape(N // BN, 1, BN)
    invc3 = dv[1].reshape(N // BN, 1, BN)

    disae = jnp.concatenate(
        [dis[:, None], ae_idx.astype(F32)[:, None],
         jnp.zeros((N, 126), F32)], axis=1)
    g2 = _make_gather_tab(128)(disae, col)
    disc3 = g2[:, 0].reshape(E // BE, 1, BE)
    cls3 = g2[:, 1].astype(I32).reshape(E // BE, 1, BE)


    layers = [
        (W_lin_0, b_lin_0, W_enc_0, b_enc_0, W_dec_0, b_dec_0, 256),
        (W_lin_1, b_lin_1, W_enc_1, b_enc_1, W_dec_1, b_dec_1, 256),
        (W_lin_2, b_lin_2, W_enc_2, b_enc_2, W_dec_2, b_dec_2, 40),
    ]

    h_prev = None
    sT_prev = None
    out40 = None
    for li, (Wl, bl, We, be, Wd, bd, dout) in enumerate(layers):
        WlT = Wl.T
        bl2 = bl.reshape(1, dout)
        WencF = We.transpose(1, 0, 2).reshape(dout, NC * CODE)
        dpad = 256 if dout == 256 else 128
        WdT = Wd.transpose(1, 0, 2)
        if dpad != dout:
            WdT = jnp.pad(WdT, ((0, 0), (0, 0), (0, dpad - dout)))
            bdp = jnp.pad(bd, ((0, 0), (0, dpad - dout)))
        else:
            bdp = bd

        if li == 0:
            h, encd = _make_dense(True, D_IN, dout)(
                x, WlT, bl2, WencF, be, ae3, dis3)
        else:
            h, encd = _make_dense(False, D_IN, dout)(
                h_prev, sT_prev, invc3, WlT, bl2, WencF, be, ae3, dis3)

        msg = _make_gather_tab(128)(encd, row)

        if dout == 256:
            decT = _make_dec(256, True)(msg, cls3, disc3, WdT, bdp)
            sflat = _make_scatter(128, True, 128)(decT.reshape(2 * E, 128), col2d, zero_n128)
            sT_prev = sflat.reshape(2, N, 128)
            h_prev = h
        else:
            dec = _make_dec(128, False)(msg, cls3, disc3, WdT, bdp)
            s2 = _make_scatter(128, False, 128)(dec, col2d, zero_n128).reshape(2, N, 128)
            out40 = _final(h, s2, invc3)

    return out40
